# Initial kernel scaffold; baseline (speedup 1.0000x reference)
import jax
import jax.numpy as jnp
from jax.experimental import pallas as pl


def kernel(type, location, time, material, method_id, quantity, type_table, loc_table, time_table, mat_table, method_table, W_q, b_q):
    raise NotImplementedError("write your pallas kernel here")


# speedup vs baseline: 2.9245x; 2.9245x over previous
import dataclasses
import functools

import jax
import jax.numpy as jnp
from jax import lax
from jax.experimental import pallas as pl
from jax.experimental.pallas import tpu as pltpu
from jax.experimental.pallas import tpu_sc as plsc

_B, _L, _D = 4096, 200, 64
_N = _B * _L
_NC, _NS = 2, 16
_NW = _NC * _NS
_CHUNK = 128
_PER_W = _N // _NW
_NCH = _PER_W // _CHUNK
_NCHT = _N // _CHUNK


def _build_sc_kernel():
    mesh = plsc.VectorSubcoreMesh(core_axis_name="c", subcore_axis_name="s")
    cp = pltpu.CompilerParams()
    if "needs_layout_passes" in pltpu.CompilerParams.__dataclass_fields__:
        cp = dataclasses.replace(cp, needs_layout_passes=False)
    if "use_tc_tiling_on_sc" in pltpu.CompilerParams.__dataclass_fields__:
        cp = dataclasses.replace(cp, use_tc_tiling_on_sc=False)

    @functools.partial(
        pl.kernel,
        compiler_params=cp,
        out_type=jax.ShapeDtypeStruct((_N, _D), jnp.float32),
        mesh=mesh,
        scratch_types=[
            pltpu.VMEM((_CHUNK,), jnp.int32),
            pltpu.VMEM((_CHUNK,), jnp.int32),
            pltpu.VMEM((_CHUNK,), jnp.int32),
            pltpu.VMEM((_CHUNK,), jnp.int32),
            pltpu.VMEM((_CHUNK,), jnp.int32),
            pltpu.VMEM((_CHUNK,), jnp.float32),
            pltpu.VMEM((_CHUNK, _D), jnp.float32),
            pltpu.VMEM((_CHUNK, _D), jnp.float32),
            pltpu.VMEM((_CHUNK, _D), jnp.float32),
            pltpu.VMEM((_CHUNK, _D), jnp.float32),
            pltpu.VMEM((_CHUNK, _D), jnp.float32),
            pltpu.VMEM((_D,), jnp.float32),
            pltpu.VMEM((_D,), jnp.float32),
            pltpu.SemaphoreType.DMA,
        ],
    )
    def k(ti_hbm, li_hbm, mi_hbm, ai_hbm, ei_hbm, q_hbm,
          ttab, ltab, titab, mtab, etab, wq_hbm, bq_hbm, out_hbm,
          ti_v, li_v, mi_v, ai_v, ei_v, q_v,
          tb, lb, mb, ab, eb, wq_v, bq_v, sem):
        wid = lax.axis_index("s") * _NC + lax.axis_index("c")
        pltpu.sync_copy(wq_hbm, wq_v)
        pltpu.sync_copy(bq_hbm, bq_v)
        wq = [wq_v[pl.ds(i * 16, 16)] for i in range(4)]
        bq = [bq_v[pl.ds(i * 16, 16)] for i in range(4)]

        @pl.loop(0, _NCH)
        def _(c):
            ch = wid * _NCH + c
            base = ch * _CHUNK
            c0 = pltpu.async_copy(ti_hbm.at[ch], ti_v, sem)
            c1 = pltpu.async_copy(li_hbm.at[ch], li_v, sem)
            c2 = pltpu.async_copy(mi_hbm.at[ch], mi_v, sem)
            c3 = pltpu.async_copy(ai_hbm.at[ch], ai_v, sem)
            c4 = pltpu.async_copy(ei_hbm.at[ch], ei_v, sem)
            c5 = pltpu.async_copy(q_hbm.at[ch], q_v, sem)
            c0.wait(); c1.wait(); c2.wait(); c3.wait(); c4.wait(); c5.wait()
            g0 = pltpu.async_copy(ttab.at[ti_v], tb, sem)
            g1 = pltpu.async_copy(ltab.at[li_v], lb, sem)
            g2 = pltpu.async_copy(titab.at[mi_v], mb, sem)
            g3 = pltpu.async_copy(mtab.at[ai_v], ab, sem)
            g4 = pltpu.async_copy(etab.at[ei_v], eb, sem)
            g0.wait(); g1.wait(); g2.wait(); g3.wait(); g4.wait()

            @pl.loop(0, _CHUNK)
            def _(t):
                tv = lax.broadcast(t, (16,))
                q = plsc.load_gather(q_v, [tv])
                for dd in range(4):
                    sl = pl.ds(dd * 16, 16)
                    ab[t, sl] = (ab[t, sl] + tb[t, sl] + lb[t, sl]
                                 + mb[t, sl] + eb[t, sl]
                                 + q * wq[dd] + bq[dd])

            pltpu.sync_copy(ab, out_hbm.at[pl.ds(base, _CHUNK)])

    return k


_sc_embed = _build_sc_kernel()


def kernel(type, location, time, material, method_id, quantity,
           type_table, loc_table, time_table, mat_table, method_table,
           W_q, b_q):
    shp = (_NCHT, _CHUNK)
    out = _sc_embed(
        type.reshape(shp), location.reshape(shp), time.reshape(shp),
        material.reshape(shp), method_id.reshape(shp),
        quantity.reshape(shp),
        type_table, loc_table, time_table, mat_table, method_table,
        W_q, b_q)
    return out.reshape(_B, _L, _D)


# speedup vs baseline: 2.9273x; 1.0010x over previous
import dataclasses
import functools

import jax
import jax.numpy as jnp
from jax import lax
from jax.experimental import pallas as pl
from jax.experimental.pallas import tpu as pltpu
from jax.experimental.pallas import tpu_sc as plsc

_B, _L, _D = 4096, 200, 64
_N = _B * _L
_NC, _NS = 2, 16
_NW = _NC * _NS
_CHUNK = 128
_PER_W = _N // _NW
_NCH = _PER_W // _CHUNK
_NCHT = _N // _CHUNK


def _build_sc_kernel():
    mesh = plsc.VectorSubcoreMesh(core_axis_name="c", subcore_axis_name="s")
    cp = pltpu.CompilerParams()
    if "needs_layout_passes" in pltpu.CompilerParams.__dataclass_fields__:
        cp = dataclasses.replace(cp, needs_layout_passes=False)
    if "use_tc_tiling_on_sc" in pltpu.CompilerParams.__dataclass_fields__:
        cp = dataclasses.replace(cp, use_tc_tiling_on_sc=False)

    scratch = []
    for _ in range(2):
        scratch += [pltpu.VMEM((_CHUNK,), jnp.int32)] * 5
        scratch += [pltpu.VMEM((_CHUNK,), jnp.float32)]
        scratch += [pltpu.VMEM((_CHUNK, _D), jnp.float32)] * 5
    scratch += [pltpu.VMEM((_D,), jnp.float32)] * 2
    scratch += [pltpu.SemaphoreType.DMA] * 6

    @functools.partial(
        pl.kernel,
        compiler_params=cp,
        out_type=jax.ShapeDtypeStruct((_N, _D), jnp.float32),
        mesh=mesh,
        scratch_types=scratch,
    )
    def k(ti_hbm, li_hbm, mi_hbm, ai_hbm, ei_hbm, q_hbm,
          ttab, ltab, titab, mtab, etab, wq_hbm, bq_hbm, out_hbm, *scr):
        idxv = [list(scr[s * 11: s * 11 + 5]) for s in (0, 1)]
        qv = [scr[s * 11 + 5] for s in (0, 1)]
        rows = [list(scr[s * 11 + 6: s * 11 + 11]) for s in (0, 1)]
        wq_v, bq_v = scr[22], scr[23]
        sem_idx, sem_g, sem_out = scr[24:26], scr[26:28], scr[28:30]

        stage_hbm = [ti_hbm, li_hbm, mi_hbm, ai_hbm, ei_hbm, q_hbm]
        tables = [ttab, ltab, titab, mtab, etab]

        wid = lax.axis_index("s") * _NC + lax.axis_index("c")
        pltpu.sync_copy(wq_hbm, wq_v)
        pltpu.sync_copy(bq_hbm, bq_v)
        wq = [wq_v[pl.ds(i * 16, 16)] for i in range(4)]
        bq = [bq_v[pl.ds(i * 16, 16)] for i in range(4)]

        def fire_idx(j, s):
            ch = wid * _NCH + j
            for hbm, v in zip(stage_hbm, idxv[s] + [qv[s]]):
                pltpu.async_copy(hbm.at[ch], v, sem_idx[s])

        def wait_idx(s):
            for hbm, v in zip(stage_hbm, idxv[s] + [qv[s]]):
                pltpu.make_async_copy(hbm.at[0], v, sem_idx[s]).wait()

        def fire_gathers(s):
            for tab, iv, buf in zip(tables, idxv[s], rows[s]):
                pltpu.async_copy(tab.at[iv], buf, sem_g[s])

        def wait_gathers(s):
            for tab, iv, buf in zip(tables, idxv[s], rows[s]):
                pltpu.make_async_copy(tab.at[iv], buf, sem_g[s]).wait()

        def fire_out(j, s):
            base = (wid * _NCH + j) * _CHUNK
            pltpu.async_copy(rows[s][3], out_hbm.at[pl.ds(base, _CHUNK)],
                             sem_out[s])

        def wait_out(s):
            pltpu.make_async_copy(rows[s][3], out_hbm.at[pl.ds(0, _CHUNK)],
                                  sem_out[s]).wait()

        def compute(s):
            tb, lb, mb, ab, eb = rows[s]
            qvv = qv[s]

            @pl.loop(0, _CHUNK, unroll=8)
            def _(t):
                q = plsc.load_gather(qvv, [lax.broadcast(t, (16,))])
                for dd in range(4):
                    sl = pl.ds(dd * 16, 16)
                    ab[t, sl] = (ab[t, sl] + tb[t, sl] + lb[t, sl]
                                 + mb[t, sl] + eb[t, sl]
                                 + q * wq[dd] + bq[dd])

        def phase(j, p, first=False):
            q = 1 - p
            wait_idx(q)
            if not first:
                wait_out(q)
            fire_gathers(q)
            wait_gathers(p)
            compute(p)
            fire_out(j, p)
            fire_idx(jnp.minimum(j + 2, _NCH - 1), p)

        fire_idx(0, 0)
        wait_idx(0)
        fire_gathers(0)
        fire_idx(1, 1)
        phase(0, 0, first=True)

        @pl.loop(1, _NCH - 1, step=2)
        def _(c):
            phase(c, 1)
            phase(c + 1, 0)

        wait_gathers(1)
        compute(1)
        fire_out(_NCH - 1, 1)
        wait_idx(0)
        wait_out(0)
        wait_out(1)

    return k


_sc_embed = _build_sc_kernel()


def kernel(type, location, time, material, method_id, quantity,
           type_table, loc_table, time_table, mat_table, method_table,
           W_q, b_q):
    shp = (_NCHT, _CHUNK)
    out = _sc_embed(
        type.reshape(shp), location.reshape(shp), time.reshape(shp),
        material.reshape(shp), method_id.reshape(shp),
        quantity.reshape(shp),
        type_table, loc_table, time_table, mat_table, method_table,
        W_q, b_q)
    return out.reshape(_B, _L, _D)


# speedup vs baseline: 7.6906x; 2.6272x over previous
import dataclasses
import functools

import jax
import jax.numpy as jnp
from jax import lax
from jax.experimental import pallas as pl
from jax.experimental.pallas import tpu as pltpu
from jax.experimental.pallas import tpu_sc as plsc

_B, _L, _D = 4096, 200, 64
_N = _B * _L
_NC, _NS = 2, 16
_NW = _NC * _NS
_CHUNK = 128
_PER_W = _N // _NW
_NCH = _PER_W // _CHUNK
_NCHT = _N // _CHUNK
_NT, _NLOC, _NTIME, _NMETH = 9, 1000, 365, 100


def _build_sc_kernel():
    mesh = plsc.VectorSubcoreMesh(core_axis_name="c", subcore_axis_name="s")
    cp = pltpu.CompilerParams()
    if "needs_layout_passes" in pltpu.CompilerParams.__dataclass_fields__:
        cp = dataclasses.replace(cp, needs_layout_passes=False)
    if "use_tc_tiling_on_sc" in pltpu.CompilerParams.__dataclass_fields__:
        cp = dataclasses.replace(cp, use_tc_tiling_on_sc=False)

    scratch = []
    for _ in range(2):
        scratch += [pltpu.VMEM((_CHUNK,), jnp.int32)] * 5
        scratch += [pltpu.VMEM((_CHUNK,), jnp.float32)]
        scratch += [pltpu.VMEM((_CHUNK, _D), jnp.float32)]
    scratch += [
        pltpu.VMEM((_NT, _D), jnp.float32),
        pltpu.VMEM((_NLOC, _D), jnp.float32),
        pltpu.VMEM((_NTIME, _D), jnp.float32),
        pltpu.VMEM((_NMETH, _D), jnp.float32),
        pltpu.VMEM((_D,), jnp.float32),
        pltpu.VMEM((_D,), jnp.float32),
    ]
    scratch += [pltpu.SemaphoreType.DMA] * 6

    @functools.partial(
        pl.kernel,
        compiler_params=cp,
        out_type=jax.ShapeDtypeStruct((_N, _D), jnp.float32),
        mesh=mesh,
        scratch_types=scratch,
    )
    def k(ti_hbm, li_hbm, mi_hbm, ai_hbm, ei_hbm, q_hbm,
          ttab, ltab, titab, mtab, etab, wq_hbm, bq_hbm, out_hbm, *scr):
        idxv = [list(scr[s * 7: s * 7 + 5]) for s in (0, 1)]
        qv = [scr[s * 7 + 5] for s in (0, 1)]
        matb = [scr[s * 7 + 6] for s in (0, 1)]
        tres, lres, mres, eres, wq_v, bq_v = scr[14:20]
        sem_idx, sem_g, sem_out = scr[20:22], scr[22:24], scr[24:26]

        stage_hbm = [ti_hbm, li_hbm, mi_hbm, ai_hbm, ei_hbm, q_hbm]

        wid = lax.axis_index("s") * _NC + lax.axis_index("c")
        pltpu.sync_copy(ttab, tres)
        pltpu.sync_copy(ltab, lres)
        pltpu.sync_copy(titab, mres)
        pltpu.sync_copy(etab, eres)
        pltpu.sync_copy(wq_hbm, wq_v)
        pltpu.sync_copy(bq_hbm, bq_v)
        wq = [wq_v[pl.ds(i * 16, 16)] for i in range(4)]
        bq = [bq_v[pl.ds(i * 16, 16)] for i in range(4)]

        def fire_idx(j, s):
            ch = wid * _NCH + j
            for hbm, v in zip(stage_hbm, idxv[s] + [qv[s]]):
                pltpu.async_copy(hbm.at[ch], v, sem_idx[s])

        def wait_idx(s):
            for hbm, v in zip(stage_hbm, idxv[s] + [qv[s]]):
                pltpu.make_async_copy(hbm.at[0], v, sem_idx[s]).wait()

        def fire_gather(s):
            pltpu.async_copy(mtab.at[idxv[s][3]], matb[s], sem_g[s])

        def wait_gather(s):
            pltpu.make_async_copy(mtab.at[idxv[s][3]], matb[s],
                                  sem_g[s]).wait()

        def fire_out(j, s):
            base = (wid * _NCH + j) * _CHUNK
            pltpu.async_copy(matb[s], out_hbm.at[pl.ds(base, _CHUNK)],
                             sem_out[s])

        def wait_out(s):
            pltpu.make_async_copy(matb[s], out_hbm.at[pl.ds(0, _CHUNK)],
                                  sem_out[s]).wait()

        def compute(s):
            tiv, liv, miv, aiv, eiv = idxv[s]
            qvv = qv[s]
            ab = matb[s]

            @pl.loop(0, _CHUNK // 16)
            def _(g):
                gb = g * 16
                sl16 = pl.ds(gb, 16)
                tvec = tiv[sl16]
                lvec = liv[sl16]
                mvec = miv[sl16]
                evec = eiv[sl16]
                qvec = qvv[sl16]
                for kk in range(16):
                    t = gb + kk
                    it, il = tvec[kk], lvec[kk]
                    im, ie = mvec[kk], evec[kk]
                    q = lax.broadcast(qvec[kk], (16,))
                    for dd in range(4):
                        sl = pl.ds(dd * 16, 16)
                        ab[t, sl] = (ab[t, sl] + tres[it, sl] + lres[il, sl]
                                     + mres[im, sl] + eres[ie, sl]
                                     + q * wq[dd] + bq[dd])

        def phase(j, p, first=False):
            q = 1 - p
            wait_idx(q)
            if not first:
                wait_out(q)
            fire_gather(q)
            wait_gather(p)
            compute(p)
            fire_out(j, p)
            fire_idx(jnp.minimum(j + 2, _NCH - 1), p)

        fire_idx(0, 0)
        wait_idx(0)
        fire_gather(0)
        fire_idx(1, 1)
        phase(0, 0, first=True)

        @pl.loop(1, _NCH - 1, step=2)
        def _(c):
            phase(c, 1)
            phase(c + 1, 0)

        wait_gather(1)
        compute(1)
        fire_out(_NCH - 1, 1)
        wait_idx(0)
        wait_out(0)
        wait_out(1)

    return k


_sc_embed = _build_sc_kernel()


def kernel(type, location, time, material, method_id, quantity,
           type_table, loc_table, time_table, mat_table, method_table,
           W_q, b_q):
    shp = (_NCHT, _CHUNK)
    out = _sc_embed(
        type.reshape(shp), location.reshape(shp), time.reshape(shp),
        material.reshape(shp), method_id.reshape(shp),
        quantity.reshape(shp),
        type_table, loc_table, time_table, mat_table, method_table,
        W_q, b_q)
    return out.reshape(_B, _L, _D)
